# baseline (speedup 1.0000x reference)
import functools

import jax
import jax.numpy as jnp
from jax import lax
from jax.experimental import pallas as pl
from jax.experimental.pallas import tpu as pltpu
from jax.experimental.pallas import tpu_sc as plsc

_N_SKELS = 100000
_N_TRAIN = 80000
_N_TOKENS = 8
_CODE_DIM = 64
_BATCH = 16384
_HALF = _BATCH // 2

_NC = 2
_NS = 16
_NW = _NC * _NS
_N_PLANES = _N_TOKENS * _CODE_DIM
_P_PER_W = _N_PLANES // _NW
_LANES = 16
_SPLIT = 50048
_NHI = _N_SKELS - _SPLIT


def _make_kernel():
    mesh = plsc.VectorSubcoreMesh(core_axis_name="c", subcore_axis_name="s")

    @functools.partial(
        pl.kernel,
        out_type=jax.ShapeDtypeStruct((_N_TOKENS, _CODE_DIM, _BATCH),
                                      jnp.float32),
        mesh=mesh,
        compiler_params=pltpu.CompilerParams(needs_layout_passes=False),
        scratch_types=[
            pltpu.VMEM((_SPLIT,), jnp.float32),
            pltpu.VMEM((_NHI,), jnp.float32),
            pltpu.VMEM((_HALF,), jnp.int32),
            pltpu.VMEM((_BATCH,), jnp.float32),
            pltpu.SemaphoreType.DMA,
            pltpu.SemaphoreType.DMA,
            pltpu.SemaphoreType.DMA,
        ],
    )
    def k(idx_hbm, zm_hbm, off_hbm, out_hbm, lo_v, hi_v, ids_v, out_v,
          sem_lo, sem_hi, sem_out):
        wid = lax.axis_index("s") * _NC + lax.axis_index("c")
        pid0 = wid * _P_PER_W

        def tc(pid):
            return pid // _CODE_DIM, pid % _CODE_DIM

        def lo_slice(tbl, pid):
            t, c = tc(pid)
            return tbl.at[t, c, pl.ds(0, _SPLIT)]

        def hi_slice(tbl, pid):
            t, c = tc(pid)
            return tbl.at[t, c, pl.ds(_SPLIT, _NHI)]

        pltpu.async_copy(lo_slice(zm_hbm, pid0), lo_v, sem_lo)
        pltpu.async_copy(hi_slice(zm_hbm, pid0), hi_v, sem_hi)
        pltpu.sync_copy(idx_hbm.at[pl.ds(0, _HALF)], ids_v)

        def ld_ids(h):
            pltpu.sync_copy(idx_hbm.at[pl.ds(h * _HALF, _HALF)], ids_v)

        def zm_pass(half_ref, base, h):
            @plsc.parallel_loop(0, _HALF, step=_LANES, unroll=8)
            def _(i):
                ids16 = ids_v[pl.ds(i, _LANES)]
                inr = (ids16 >= base) & (ids16 < base + half_ref.shape[0])
                vals = plsc.load_gather(half_ref, [ids16 - base], mask=inr)
                osl = pl.ds(h * _HALF + i, _LANES)
                out_v[osl] = jnp.where(inr, vals, out_v[osl])

        def off_pass(half_ref, base, h):
            @plsc.parallel_loop(0, _HALF, step=_LANES, unroll=8)
            def _(i):
                ids16 = ids_v[pl.ds(i, _LANES)]
                inr = (ids16 >= base) & (ids16 < base + half_ref.shape[0])
                vals = plsc.load_gather(half_ref, [ids16 - base], mask=inr)
                mvec = jnp.where(ids16 < _N_TRAIN, jnp.float32(1.0),
                                 jnp.float32(0.0))
                osl = pl.ds(h * _HALF + i, _LANES)
                cand = out_v[osl] + vals * mvec
                out_v[osl] = jnp.where(inr, cand, out_v[osl])

        def plane_body(p, carry):
            pid = pid0 + p
            t, c = tc(pid)

            pltpu.make_async_copy(lo_slice(zm_hbm, pid), lo_v, sem_lo).wait()

            @pl.when(p > 0)
            def _drain_out():
                tp, cp_ = tc(pid - 1)
                pltpu.make_async_copy(out_v, out_hbm.at[tp, cp_],
                                      sem_out).wait()

            zm_pass(lo_v, 0, 0)
            ld_ids(1)
            zm_pass(lo_v, 0, 1)
            pltpu.async_copy(lo_slice(off_hbm, pid), lo_v, sem_lo)

            pltpu.make_async_copy(hi_slice(zm_hbm, pid), hi_v, sem_hi).wait()
            zm_pass(hi_v, _SPLIT, 1)
            ld_ids(0)
            zm_pass(hi_v, _SPLIT, 0)
            pltpu.async_copy(hi_slice(off_hbm, pid), hi_v, sem_hi)

            pltpu.make_async_copy(lo_slice(off_hbm, pid), lo_v, sem_lo).wait()
            off_pass(lo_v, 0, 0)
            ld_ids(1)
            off_pass(lo_v, 0, 1)

            @pl.when(p < _P_PER_W - 1)
            def _next_lo():
                pltpu.async_copy(lo_slice(zm_hbm, pid + 1), lo_v, sem_lo)

            pltpu.make_async_copy(hi_slice(off_hbm, pid), hi_v, sem_hi).wait()
            off_pass(hi_v, _SPLIT, 1)
            ld_ids(0)
            off_pass(hi_v, _SPLIT, 0)

            @pl.when(p < _P_PER_W - 1)
            def _next_hi():
                pltpu.async_copy(hi_slice(zm_hbm, pid + 1), hi_v, sem_hi)

            pltpu.async_copy(out_v, out_hbm.at[t, c], sem_out)
            return carry

        lax.fori_loop(0, _P_PER_W, plane_body, 0, unroll=False)
        tl, cl = tc(pid0 + _P_PER_W - 1)
        pltpu.make_async_copy(out_v, out_hbm.at[tl, cl], sem_out).wait()

    return k


_kernel_call = _make_kernel()


@jax.jit
def kernel(tgt_skel_id, z_means, offset):
    zm_t = jnp.transpose(z_means, (1, 2, 0))
    off_t = jnp.transpose(offset, (1, 2, 0))
    out_t = _kernel_call(tgt_skel_id, zm_t, off_t)
    return jnp.transpose(out_t, (2, 0, 1))


# speedup vs baseline: 1.0309x; 1.0309x over previous
import functools

import jax
import jax.numpy as jnp
from jax import lax
from jax.experimental import pallas as pl
from jax.experimental.pallas import tpu as pltpu
from jax.experimental.pallas import tpu_sc as plsc

_N_SKELS = 100000
_N_TRAIN = 80000
_N_TOKENS = 8
_CODE_DIM = 64
_BATCH = 16384
_HALF = _BATCH // 2

_NC = 2
_NS = 16
_NW = _NC * _NS
_N_PLANES = _N_TOKENS * _CODE_DIM
_P_PER_W = _N_PLANES // _NW
_LANES = 16


def _make_kernel():
    mesh = plsc.VectorSubcoreMesh(core_axis_name="c", subcore_axis_name="s")

    @functools.partial(
        pl.kernel,
        out_type=jax.ShapeDtypeStruct((_N_TOKENS, _CODE_DIM, _BATCH),
                                      jnp.float32),
        mesh=mesh,
        compiler_params=pltpu.CompilerParams(needs_layout_passes=False),
        scratch_types=[
            pltpu.VMEM((_N_SKELS,), jnp.float32),
            pltpu.VMEM((_HALF,), jnp.int32),
            pltpu.VMEM((_BATCH,), jnp.float32),
            pltpu.SemaphoreType.DMA,
            pltpu.SemaphoreType.DMA,
        ],
    )
    def k(idx_hbm, zm_hbm, off_hbm, out_hbm, plane_v, ids_v, out_v, sem,
          sem_out):
        wid = lax.axis_index("s") * _NC + lax.axis_index("c")
        pltpu.sync_copy(idx_hbm.at[pl.ds(0, _HALF)], ids_v)

        def plane_body(p, carry):
            pid = wid * _P_PER_W + p
            t = pid // _CODE_DIM
            c = pid % _CODE_DIM

            pltpu.sync_copy(zm_hbm.at[t, c], plane_v)
            @pl.when(p > 0)
            def _drain():
                tp = (pid - 1) // _CODE_DIM
                cp_ = (pid - 1) % _CODE_DIM
                pltpu.make_async_copy(out_v, out_hbm.at[tp, cp_],
                                      sem_out).wait()

            def zm_pass(h):
                @plsc.parallel_loop(0, _HALF, step=_LANES, unroll=16)
                def _(i):
                    ids16 = ids_v[pl.ds(i, _LANES)]
                    vals = plsc.load_gather(plane_v, [ids16])
                    out_v[pl.ds(h * _HALF + i, _LANES)] = vals

            zm_pass(0)
            pltpu.sync_copy(idx_hbm.at[pl.ds(_HALF, _HALF)], ids_v)
            zm_pass(1)

            pltpu.sync_copy(off_hbm.at[t, c], plane_v)

            def off_pass(h):
                @plsc.parallel_loop(0, _HALF, step=_LANES, unroll=16)
                def _(i):
                    ids16 = ids_v[pl.ds(i, _LANES)]
                    vals = plsc.load_gather(plane_v, [ids16])
                    mvec = jnp.where(ids16 < _N_TRAIN, jnp.float32(1.0),
                                     jnp.float32(0.0))
                    osl = pl.ds(h * _HALF + i, _LANES)
                    out_v[osl] = out_v[osl] + vals * mvec

            off_pass(1)
            pltpu.sync_copy(idx_hbm.at[pl.ds(0, _HALF)], ids_v)
            off_pass(0)

            pltpu.async_copy(out_v, out_hbm.at[t, c], sem_out)
            return carry

        lax.fori_loop(0, _P_PER_W, plane_body, 0, unroll=False)
        last = wid * _P_PER_W + _P_PER_W - 1
        pltpu.make_async_copy(
            out_v, out_hbm.at[last // _CODE_DIM, last % _CODE_DIM],
            sem_out).wait()

    return k


_kernel_call = _make_kernel()


@jax.jit
def kernel(tgt_skel_id, z_means, offset):
    zm_t = jnp.transpose(z_means, (1, 2, 0))
    off_t = jnp.transpose(offset, (1, 2, 0))
    out_t = _kernel_call(tgt_skel_id, zm_t, off_t)
    return jnp.transpose(out_t, (2, 0, 1))
